# Initial kernel scaffold; baseline (speedup 1.0000x reference)
import jax
import jax.numpy as jnp
from jax.experimental import pallas as pl


def kernel(node_fts, gkt_edge_fts, hidden, cfg_indices_padded, gkt_indices_padded, W_m, b_m, W_skip, b_skip, W_a1, b_a1, W_a2, b_a2, W_ae, b_ae):
    raise NotImplementedError("write your pallas kernel here")


# speedup vs baseline: 42.9087x; 42.9087x over previous
import functools

import jax
import jax.numpy as jnp
import numpy as np
from jax import lax
from jax.experimental import pallas as pl
from jax.experimental.pallas import tpu as pltpu
from jax.experimental.pallas import tpu_sc as plsc

NB_HEADS = 8
HEAD_SIZE = 16
OUT_SIZE = NB_HEADS * HEAD_SIZE
LANES = 16
NC = 2
NS = 16
NW = NC * NS
ROW_W = 16 + OUT_SIZE


def _dot(a, b):
    return jax.lax.dot(a, b, precision=jax.lax.Precision.HIGHEST,
                       preferred_element_type=jnp.float32)


def _dense1_body(nf, hid, wa_nf, wa_h, ba, wm_nf, wm_h, bm, ws_nf, ws_h, bs,
                 att_o, v_o, sk_o):
    a = nf[...]
    h = hid[...]
    att_o[...] = _dot(a, wa_nf[...]) + _dot(h, wa_h[...]) + ba[...]
    v_o[...] = _dot(a, wm_nf[...]) + _dot(h, wm_h[...]) + bm[...]
    sk_o[...] = _dot(a, ws_nf[...]) + _dot(h, ws_h[...]) + bs[...]


def _dense1(node_fts, hidden, wa_nf, wa_h, ba, wm_nf, wm_h, bm, ws_nf, ws_h, bs):
    n, d = node_fts.shape
    r = _row_block(n)
    grid = (n // r,)
    full = lambda shape: pl.BlockSpec(shape, lambda i: (0,) * len(shape))
    rows = lambda w: pl.BlockSpec((r, w), lambda i: (i, 0))
    return pl.pallas_call(
        _dense1_body,
        grid=grid,
        in_specs=[rows(128), rows(128), full((128, 16)), full((128, 16)),
                  full((1, 16)), full((128, 128)), full((128, 128)),
                  full((1, 128)), full((128, 128)), full((128, 128)),
                  full((1, 128))],
        out_specs=[rows(16), rows(128), rows(128)],
        out_shape=[jax.ShapeDtypeStruct((n, 16), jnp.float32),
                   jax.ShapeDtypeStruct((n, 128), jnp.float32),
                   jax.ShapeDtypeStruct((n, 128), jnp.float32)],
    )(node_fts, hidden, wa_nf, wa_h, ba, wm_nf, wm_h, bm, ws_nf, ws_h, bs)


def _row_block(n):
    for cand in (2000, 1000, 500, 250, 200, 100, 50, 25, 10, 8, 5, 4, 2):
        if n % cand == 0:
            return cand
    return n


def _ae_body(ef, w, b, o):
    o[...] = _dot(ef[...], w[...]) + b[...]


def _edge_att(gkt_edge_fts, wbig, b128):
    e = gkt_edge_fts.shape[0]
    assert e % 8 == 0
    ep = e // 8
    ef_packed = gkt_edge_fts.reshape(ep, 128)
    r = 5000 if ep % 5000 == 0 else _row_block(ep)
    out = pl.pallas_call(
        _ae_body,
        grid=(ep // r,),
        in_specs=[pl.BlockSpec((r, 128), lambda i: (i, 0)),
                  pl.BlockSpec((128, 128), lambda i: (0, 0)),
                  pl.BlockSpec((1, 128), lambda i: (0, 0))],
        out_specs=pl.BlockSpec((r, 128), lambda i: (i, 0)),
        out_shape=jax.ShapeDtypeStruct((ep, 128), jnp.float32),
    )(ef_packed, wbig, b128)
    return out.reshape(e, 16)


def _combine2_body(sc, nf, sk1, ksel, kacc, wa_nf, wa_h, ba, wm_nf, wm_h, bm,
                   ws_nf, ws_h, bs, att_o, v_o, sk_o):
    both = sc[0] + sc[1]
    s_rep = _dot(both, ksel[...])
    acc = _dot(both, kacc[...])
    ch = jnp.maximum(acc / jnp.maximum(s_rep, 1e-30) + sk1[...], 0.0)
    a = nf[...]
    att_o[...] = _dot(a, wa_nf[...]) + _dot(ch, wa_h[...]) + ba[...]
    v_o[...] = _dot(a, wm_nf[...]) + _dot(ch, wm_h[...]) + bm[...]
    sk_o[...] = _dot(a, ws_nf[...]) + _dot(ch, ws_h[...]) + bs[...]


def _combine2(sc_out, node_fts, sk1, ksel, kacc, wa_nf, wa_h, ba, wm_nf, wm_h,
              bm, ws_nf, ws_h, bs):
    n = node_fts.shape[0]
    r = _row_block(n)
    grid = (n // r,)
    full = lambda shape: pl.BlockSpec(shape, lambda i: (0,) * len(shape))
    rows = lambda w: pl.BlockSpec((r, w), lambda i: (i, 0))
    scs = pl.BlockSpec((2, r, ROW_W), lambda i: (0, i, 0))
    return pl.pallas_call(
        _combine2_body,
        grid=grid,
        in_specs=[scs, rows(128), rows(128), full((ROW_W, 128)),
                  full((ROW_W, 128)), full((128, 16)), full((128, 16)),
                  full((1, 16)), full((128, 128)), full((128, 128)),
                  full((1, 128)), full((128, 128)), full((128, 128)),
                  full((1, 128))],
        out_specs=[rows(16), rows(128), rows(128)],
        out_shape=[jax.ShapeDtypeStruct((n, 16), jnp.float32),
                   jax.ShapeDtypeStruct((n, 128), jnp.float32),
                   jax.ShapeDtypeStruct((n, 128), jnp.float32)],
    )(sc_out, node_fts, sk1, ksel, kacc, wa_nf, wa_h, ba, wm_nf, wm_h, bm,
      ws_nf, ws_h, bs)


def _final_body(sc, sk2, ksel, kacc, o):
    both = sc[0] + sc[1]
    s_rep = _dot(both, ksel[...])
    acc = _dot(both, kacc[...])
    o[...] = jnp.maximum(acc / jnp.maximum(s_rep, 1e-30) + sk2[...], 0.0)


def _final(sc_out, sk2, ksel, kacc):
    n = sk2.shape[0]
    r = _row_block(n)
    return pl.pallas_call(
        _final_body,
        grid=(n // r,),
        in_specs=[pl.BlockSpec((2, r, ROW_W), lambda i: (0, i, 0)),
                  pl.BlockSpec((r, 128), lambda i: (i, 0)),
                  pl.BlockSpec((ROW_W, 128), lambda i: (0, 0)),
                  pl.BlockSpec((ROW_W, 128), lambda i: (0, 0))],
        out_specs=pl.BlockSpec((r, 128), lambda i: (i, 0)),
        out_shape=jax.ShapeDtypeStruct((n, 128), jnp.float32),
    )(sc_out, sk2, ksel, kacc)


def _pick_chunk(per_w):
    for cand in (128, 120, 112, 104, 96, 88, 80, 72, 64, 56, 48, 40, 32, 24, 16, 8):
        if per_w % cand == 0:
            return cand
    return per_w


def _dyn_gather(x, idx):
    dnums = jax.lax.GatherDimensionNumbers(
        offset_dims=(), collapsed_slice_dims=(0,), start_index_map=(0,))
    return jax.lax.gather(x, idx[:, None], dnums, slice_sizes=(1,),
                          mode=jax.lax.GatherScatterMode.PROMISE_IN_BOUNDS)


def _edge_layer_sc(att, vals, rows, cols, ae):
    e_total = rows.shape[0]
    n = att.shape[0]
    assert e_total % NW == 0
    per_w = e_total // NW
    c = _pick_chunk(per_w)
    n_chunks = per_w // c
    zc = _pick_chunk(n)
    assert zc <= c
    n_zchunks = n // zc
    has_ae = ae is not None

    mesh = plsc.VectorSubcoreMesh(core_axis_name="c", subcore_axis_name="s")

    scratch = [
        pltpu.VMEM((c,), jnp.int32),
        pltpu.VMEM((c,), jnp.int32),
        pltpu.VMEM((c, 16), jnp.float32),
        pltpu.VMEM((c, 16), jnp.float32),
        pltpu.VMEM((c, 128), jnp.float32),
        pltpu.VMEM((c, ROW_W), jnp.float32),
        pltpu.VMEM_SHARED((n, ROW_W), jnp.float32),
        pltpu.SemaphoreType.DMA,
        pltpu.SemaphoreType.DMA,
        pltpu.SemaphoreType.DMA,
    ]
    if has_ae:
        scratch.insert(6, pltpu.VMEM((c, 16), jnp.float32))

    def body(*refs):
        if has_ae:
            (att_h, v_h, rows_h, cols_h, ae_h, out_h, rowv, colv, attr, attc,
             vbuf, pv, aebuf, sh, sem1, sem2, sem3) = refs
        else:
            (att_h, v_h, rows_h, cols_h, out_h, rowv, colv, attr, attc,
             vbuf, pv, sh, sem1, sem2, sem3) = refs
        cid = lax.axis_index("c")
        sid = lax.axis_index("s")
        wid = cid * NS + sid
        base_e = wid * per_w

        zeros16 = jnp.zeros((LANES,), jnp.float32)

        @pl.loop(0, min(c, zc))
        def _zrow(i):
            for hsub in range(ROW_W // LANES):
                pv[i, pl.ds(hsub * LANES, LANES)] = zeros16

        @pl.loop(sid, n_zchunks, step=NS)
        def _zfill(k):
            pltpu.sync_copy(pv.at[pl.ds(0, zc)], sh.at[pl.ds(k * zc, zc)])

        plsc.subcore_barrier()

        rot8 = jax.lax.iota(jnp.int32, LANES) ^ 8

        @pl.loop(0, n_chunks)
        def _chunk(ci):
            eoff = base_e + ci * c
            pltpu.sync_copy(rows_h.at[pl.ds(eoff, c)], rowv)
            pltpu.sync_copy(cols_h.at[pl.ds(eoff, c)], colv)
            g1 = pltpu.make_async_copy(att_h.at[rowv], attr, sem1)
            g1.start()
            g2 = pltpu.make_async_copy(att_h.at[colv], attc, sem2)
            g2.start()
            g3 = pltpu.make_async_copy(v_h.at[colv], vbuf, sem3)
            g3.start()
            if has_ae:
                pltpu.sync_copy(ae_h.at[pl.ds(eoff, c)], aebuf)
            g1.wait()
            g2.wait()
            g3.wait()

            @pl.loop(0, c)
            def _edge(ei):
                a1 = attr[ei, :]
                a2 = attc[ei, :]
                logit = a1 + _dyn_gather(a2, rot8)
                if has_ae:
                    logit = logit + aebuf[ei, :]
                p = jnp.exp(jnp.maximum(logit, logit * 0.01))
                pv[ei, pl.ds(0, LANES)] = p
                vals_row_base = 16
                for h in range(NB_HEADS):
                    mult = _dyn_gather(p, jnp.full((LANES,), h, jnp.int32))
                    vv = vbuf[ei, pl.ds(h * LANES, LANES)]
                    pv[ei, pl.ds(vals_row_base + h * LANES, LANES)] = vv * mult

            pltpu.sync_copy(pv, sh.at[rowv], add=True)

        plsc.subcore_barrier()

        @pl.loop(sid, n_zchunks, step=NS)
        def _dump(k):
            pltpu.sync_copy(sh.at[pl.ds(k * zc, zc)],
                            out_h.at[cid, pl.ds(k * zc, zc)])

    fn = pl.kernel(
        body,
        out_type=jax.ShapeDtypeStruct((NC, n, ROW_W), jnp.float32),
        mesh=mesh,
        scratch_types=scratch,
        compiler_params=pltpu.CompilerParams(use_tc_tiling_on_sc=False),
    )
    args = (att, vals, rows, cols) + ((ae,) if has_ae else ())
    return fn(*args)


def kernel(node_fts, gkt_edge_fts, hidden, cfg_indices_padded,
           gkt_indices_padded, W_m, b_m, W_skip, b_skip, W_a1, b_a1, W_a2,
           b_a2, W_ae, b_ae):
    n, d_feat = node_fts.shape

    w_att = jnp.concatenate([W_a1, W_a2], axis=1)
    wa_nf, wa_h = w_att[:d_feat], w_att[d_feat:]
    ba = jnp.concatenate([b_a1, b_a2])[None, :]
    wm_nf, wm_h = W_m[:d_feat], W_m[d_feat:]
    bm = b_m[None, :]
    ws_nf, ws_h = W_skip[:d_feat], W_skip[d_feat:]
    bs = b_skip[None, :]
    wae16 = jnp.pad(W_ae, ((0, 0), (0, 8)))
    bae16 = jnp.pad(b_ae, (0, 8))
    wbig = jnp.kron(jnp.eye(8, dtype=jnp.float32), wae16)
    bae128 = jnp.tile(bae16, 8)[None, :]

    ksel_np = np.zeros((ROW_W, 128), np.float32)
    for h in range(NB_HEADS):
        ksel_np[h, h * HEAD_SIZE:(h + 1) * HEAD_SIZE] = 1.0
    kacc_np = np.zeros((ROW_W, 128), np.float32)
    kacc_np[16:, :] = np.eye(128, dtype=np.float32)
    ksel = jnp.asarray(ksel_np)
    kacc = jnp.asarray(kacc_np)

    cfg_rows = cfg_indices_padded[:, 0]
    cfg_cols = cfg_indices_padded[:, 1]
    gkt_rows = gkt_indices_padded[:, 0]
    gkt_cols = gkt_indices_padded[:, 1]

    att1, v1, sk1 = _dense1(node_fts, hidden, wa_nf, wa_h, ba, wm_nf, wm_h,
                            bm, ws_nf, ws_h, bs)
    sc1 = _edge_layer_sc(att1, v1, cfg_rows, cfg_cols, None)

    ae = _edge_att(gkt_edge_fts, wbig, bae128)
    att2, v2, sk2 = _combine2(sc1, node_fts, sk1, ksel, kacc, wa_nf, wa_h, ba,
                              wm_nf, wm_h, bm, ws_nf, ws_h, bs)
    sc2 = _edge_layer_sc(att2, v2, gkt_rows, gkt_cols, ae)
    return _final(sc2, sk2, ksel, kacc)


# speedup vs baseline: 44.1687x; 1.0294x over previous
import functools

import jax
import jax.numpy as jnp
import numpy as np
from jax import lax
from jax.experimental import pallas as pl
from jax.experimental.pallas import tpu as pltpu
from jax.experimental.pallas import tpu_sc as plsc

NB_HEADS = 8
HEAD_SIZE = 16
OUT_SIZE = NB_HEADS * HEAD_SIZE
LANES = 16
NC = 2
NS = 16
NW = NC * NS
ROW_W = 16 + OUT_SIZE


def _dot(a, b):
    return jax.lax.dot(a, b, precision=jax.lax.Precision.HIGHEST,
                       preferred_element_type=jnp.float32)


def _dense1_body(nf, hid, wa_nf, wa_h, ba, wm_nf, wm_h, bm, ws_nf, ws_h, bs,
                 att_o, v_o, sk_o):
    a = nf[...]
    h = hid[...]
    att_o[...] = _dot(a, wa_nf[...]) + _dot(h, wa_h[...]) + ba[...]
    v_o[...] = _dot(a, wm_nf[...]) + _dot(h, wm_h[...]) + bm[...]
    sk_o[...] = _dot(a, ws_nf[...]) + _dot(h, ws_h[...]) + bs[...]


def _dense1(node_fts, hidden, wa_nf, wa_h, ba, wm_nf, wm_h, bm, ws_nf, ws_h, bs):
    n, d = node_fts.shape
    r = _row_block(n)
    grid = (n // r,)
    full = lambda shape: pl.BlockSpec(shape, lambda i: (0,) * len(shape))
    rows = lambda w: pl.BlockSpec((r, w), lambda i: (i, 0))
    return pl.pallas_call(
        _dense1_body,
        grid=grid,
        in_specs=[rows(128), rows(128), full((128, 16)), full((128, 16)),
                  full((1, 16)), full((128, 128)), full((128, 128)),
                  full((1, 128)), full((128, 128)), full((128, 128)),
                  full((1, 128))],
        out_specs=[rows(16), rows(128), rows(128)],
        out_shape=[jax.ShapeDtypeStruct((n, 16), jnp.float32),
                   jax.ShapeDtypeStruct((n, 128), jnp.float32),
                   jax.ShapeDtypeStruct((n, 128), jnp.float32)],
    )(node_fts, hidden, wa_nf, wa_h, ba, wm_nf, wm_h, bm, ws_nf, ws_h, bs)


def _row_block(n):
    for cand in (2000, 1000, 500, 250, 200, 100, 50, 25, 10, 8, 5, 4, 2):
        if n % cand == 0:
            return cand
    return n


def _ae_body(ef, w, b, o):
    o[...] = _dot(ef[...], w[...]) + b[...]


def _edge_att(gkt_edge_fts, wbig, b128):
    e = gkt_edge_fts.shape[0]
    assert e % 8 == 0
    ep = e // 8
    ef_packed = gkt_edge_fts.reshape(ep, 128)
    r = 5000 if ep % 5000 == 0 else _row_block(ep)
    out = pl.pallas_call(
        _ae_body,
        grid=(ep // r,),
        in_specs=[pl.BlockSpec((r, 128), lambda i: (i, 0)),
                  pl.BlockSpec((128, 128), lambda i: (0, 0)),
                  pl.BlockSpec((1, 128), lambda i: (0, 0))],
        out_specs=pl.BlockSpec((r, 128), lambda i: (i, 0)),
        out_shape=jax.ShapeDtypeStruct((ep, 128), jnp.float32),
    )(ef_packed, wbig, b128)
    return out.reshape(e, 16)


def _combine2_body(sc, nf, sk1, ksel, kacc, wa_nf, wa_h, ba, wm_nf, wm_h, bm,
                   ws_nf, ws_h, bs, att_o, v_o, sk_o):
    both = sc[0] + sc[1]
    s_rep = _dot(both, ksel[...])
    acc = _dot(both, kacc[...])
    ch = jnp.maximum(acc / jnp.maximum(s_rep, 1e-30) + sk1[...], 0.0)
    a = nf[...]
    att_o[...] = _dot(a, wa_nf[...]) + _dot(ch, wa_h[...]) + ba[...]
    v_o[...] = _dot(a, wm_nf[...]) + _dot(ch, wm_h[...]) + bm[...]
    sk_o[...] = _dot(a, ws_nf[...]) + _dot(ch, ws_h[...]) + bs[...]


def _combine2(sc_out, node_fts, sk1, ksel, kacc, wa_nf, wa_h, ba, wm_nf, wm_h,
              bm, ws_nf, ws_h, bs):
    n = node_fts.shape[0]
    r = _row_block(n)
    grid = (n // r,)
    full = lambda shape: pl.BlockSpec(shape, lambda i: (0,) * len(shape))
    rows = lambda w: pl.BlockSpec((r, w), lambda i: (i, 0))
    scs = pl.BlockSpec((2, r, ROW_W), lambda i: (0, i, 0))
    return pl.pallas_call(
        _combine2_body,
        grid=grid,
        in_specs=[scs, rows(128), rows(128), full((ROW_W, 128)),
                  full((ROW_W, 128)), full((128, 16)), full((128, 16)),
                  full((1, 16)), full((128, 128)), full((128, 128)),
                  full((1, 128)), full((128, 128)), full((128, 128)),
                  full((1, 128))],
        out_specs=[rows(16), rows(128), rows(128)],
        out_shape=[jax.ShapeDtypeStruct((n, 16), jnp.float32),
                   jax.ShapeDtypeStruct((n, 128), jnp.float32),
                   jax.ShapeDtypeStruct((n, 128), jnp.float32)],
    )(sc_out, node_fts, sk1, ksel, kacc, wa_nf, wa_h, ba, wm_nf, wm_h, bm,
      ws_nf, ws_h, bs)


def _final_body(sc, sk2, ksel, kacc, o):
    both = sc[0] + sc[1]
    s_rep = _dot(both, ksel[...])
    acc = _dot(both, kacc[...])
    o[...] = jnp.maximum(acc / jnp.maximum(s_rep, 1e-30) + sk2[...], 0.0)


def _final(sc_out, sk2, ksel, kacc):
    n = sk2.shape[0]
    r = _row_block(n)
    return pl.pallas_call(
        _final_body,
        grid=(n // r,),
        in_specs=[pl.BlockSpec((2, r, ROW_W), lambda i: (0, i, 0)),
                  pl.BlockSpec((r, 128), lambda i: (i, 0)),
                  pl.BlockSpec((ROW_W, 128), lambda i: (0, 0)),
                  pl.BlockSpec((ROW_W, 128), lambda i: (0, 0))],
        out_specs=pl.BlockSpec((r, 128), lambda i: (i, 0)),
        out_shape=jax.ShapeDtypeStruct((n, 128), jnp.float32),
    )(sc_out, sk2, ksel, kacc)


def _pick_chunk(per_w, cap=128):
    for cand in (128, 120, 112, 104, 96, 88, 80, 72, 64, 56, 48, 40, 32, 24, 16, 8):
        if cand <= cap and per_w % cand == 0:
            return cand
    return per_w


def _dyn_gather(x, idx):
    dnums = jax.lax.GatherDimensionNumbers(
        offset_dims=(), collapsed_slice_dims=(0,), start_index_map=(0,))
    return jax.lax.gather(x, idx[:, None], dnums, slice_sizes=(1,),
                          mode=jax.lax.GatherScatterMode.PROMISE_IN_BOUNDS)


def _edge_layer_sc(att, vals, rows, cols, ae):
    e_total = rows.shape[0]
    n = att.shape[0]
    assert e_total % NW == 0
    per_w = e_total // NW
    c = _pick_chunk(per_w, cap=40)
    n_chunks = per_w // c
    zc = _pick_chunk(n, cap=c)
    n_zchunks = n // zc
    has_ae = ae is not None

    mesh = plsc.VectorSubcoreMesh(core_axis_name="c", subcore_axis_name="s")
    nbuf = 2
    assert n_chunks >= 4 and n_chunks % nbuf == 0

    scratch = [
        [pltpu.VMEM((c,), jnp.int32)] * nbuf,
        [pltpu.VMEM((c,), jnp.int32)] * nbuf,
        [pltpu.VMEM((c, 16), jnp.float32)] * nbuf,
        [pltpu.VMEM((c, 16), jnp.float32)] * nbuf,
        [pltpu.VMEM((c, 128), jnp.float32)] * nbuf,
        [pltpu.VMEM((c, ROW_W), jnp.float32)] * nbuf,
        [pltpu.VMEM((c, 16), jnp.float32)] * nbuf,
        pltpu.VMEM_SHARED((n, ROW_W), jnp.float32),
        [pltpu.SemaphoreType.DMA] * nbuf,
        [pltpu.SemaphoreType.DMA] * nbuf,
        [pltpu.SemaphoreType.DMA] * nbuf,
        [pltpu.SemaphoreType.DMA] * nbuf,
        [pltpu.SemaphoreType.DMA] * nbuf,
    ]

    def body(*refs):
        if has_ae:
            (att_h, v_h, rows_h, cols_h, ae_h, out_h, rowv, colv, attr, attc,
             vbuf, pv, aebuf, sh, gsem1, gsem2, gsem3, aesem, ssem) = refs
        else:
            (att_h, v_h, rows_h, cols_h, out_h, rowv, colv, attr, attc,
             vbuf, pv, aebuf, sh, gsem1, gsem2, gsem3, aesem, ssem) = refs
        cid = lax.axis_index("c")
        sid = lax.axis_index("s")
        wid = cid * NS + sid
        base_e = wid * per_w

        zeros16 = jnp.zeros((LANES,), jnp.float32)

        @pl.loop(0, min(c, zc))
        def _zrow(i):
            for b in range(nbuf):
                for hsub in range(ROW_W // LANES):
                    pv[b][i, pl.ds(hsub * LANES, LANES)] = zeros16

        @pl.loop(sid, n_zchunks, step=NS)
        def _zfill(k):
            pltpu.sync_copy(pv[0].at[pl.ds(0, zc)], sh.at[pl.ds(k * zc, zc)])

        plsc.subcore_barrier()

        rot8 = jax.lax.iota(jnp.int32, LANES) ^ 8

        def load_idx(j, b):
            eoff = base_e + j * c
            pltpu.sync_copy(rows_h.at[pl.ds(eoff, c)], rowv[b])
            pltpu.sync_copy(cols_h.at[pl.ds(eoff, c)], colv[b])

        def start_gathers(j, b):
            pltpu.async_copy(att_h.at[rowv[b]], attr[b], gsem1[b])
            pltpu.async_copy(att_h.at[colv[b]], attc[b], gsem2[b])
            pltpu.async_copy(v_h.at[colv[b]], vbuf[b], gsem3[b])
            if has_ae:
                eoff = base_e + j * c
                pltpu.async_copy(ae_h.at[pl.ds(eoff, c)], aebuf[b], aesem[b])

        def wait_gathers(b):
            pltpu.make_async_copy(att_h.at[rowv[b]], attr[b], gsem1[b]).wait()
            pltpu.make_async_copy(att_h.at[colv[b]], attc[b], gsem2[b]).wait()
            pltpu.make_async_copy(v_h.at[colv[b]], vbuf[b], gsem3[b]).wait()
            if has_ae:
                pltpu.make_async_copy(ae_h.at[pl.ds(0, c)], aebuf[b],
                                      aesem[b]).wait()

        def start_scatter(b):
            pltpu.async_copy(pv[b], sh.at[rowv[b]], ssem[b], add=True)

        def wait_scatter(b):
            pltpu.make_async_copy(pv[b], sh.at[rowv[b]], ssem[b]).wait()

        def compute(b):
            @pl.loop(0, c)
            def _edge(ei):
                a1 = attr[b][ei, :]
                a2 = attc[b][ei, :]
                logit = a1 + _dyn_gather(a2, rot8)
                if has_ae:
                    logit = logit + aebuf[b][ei, :]
                p = jnp.exp(jnp.maximum(logit, logit * 0.01))
                pv[b][ei, pl.ds(0, LANES)] = p
                for h in range(NB_HEADS):
                    mult = _dyn_gather(p, jnp.full((LANES,), h, jnp.int32))
                    vv = vbuf[b][ei, pl.ds(h * LANES, LANES)]
                    pv[b][ei, pl.ds(16 + h * LANES, LANES)] = vv * mult

        load_idx(0, 0)
        load_idx(0, 1)
        start_gathers(0, 0)
        start_scatter(1)

        def step(j, b):
            nb = 1 - b
            wait_gathers(b)
            wait_scatter(nb)
            load_idx(j + 1, nb)
            start_gathers(j + 1, nb)
            compute(b)
            start_scatter(b)

        @pl.loop(0, (n_chunks - 2) // nbuf)
        def _chunk(ci):
            step(ci * nbuf, 0)
            step(ci * nbuf + 1, 1)

        step(n_chunks - 2, 0)
        wait_gathers(1)
        compute(1)
        start_scatter(1)
        wait_scatter(0)
        wait_scatter(1)

        plsc.subcore_barrier()

        @pl.loop(sid, n_zchunks, step=NS)
        def _dump(k):
            pltpu.sync_copy(sh.at[pl.ds(k * zc, zc)],
                            out_h.at[cid, pl.ds(k * zc, zc)])

    fn = pl.kernel(
        body,
        out_type=jax.ShapeDtypeStruct((NC, n, ROW_W), jnp.float32),
        mesh=mesh,
        scratch_types=scratch,
        compiler_params=pltpu.CompilerParams(use_tc_tiling_on_sc=False),
    )
    args = (att, vals, rows, cols) + ((ae,) if has_ae else ())
    return fn(*args)


def kernel(node_fts, gkt_edge_fts, hidden, cfg_indices_padded,
           gkt_indices_padded, W_m, b_m, W_skip, b_skip, W_a1, b_a1, W_a2,
           b_a2, W_ae, b_ae):
    n, d_feat = node_fts.shape

    w_att = jnp.concatenate([W_a1, W_a2], axis=1)
    wa_nf, wa_h = w_att[:d_feat], w_att[d_feat:]
    ba = jnp.concatenate([b_a1, b_a2])[None, :]
    wm_nf, wm_h = W_m[:d_feat], W_m[d_feat:]
    bm = b_m[None, :]
    ws_nf, ws_h = W_skip[:d_feat], W_skip[d_feat:]
    bs = b_skip[None, :]
    wae16 = jnp.pad(W_ae, ((0, 0), (0, 8)))
    bae16 = jnp.pad(b_ae, (0, 8))
    wbig = jnp.kron(jnp.eye(8, dtype=jnp.float32), wae16)
    bae128 = jnp.tile(bae16, 8)[None, :]

    ksel_np = np.zeros((ROW_W, 128), np.float32)
    for h in range(NB_HEADS):
        ksel_np[h, h * HEAD_SIZE:(h + 1) * HEAD_SIZE] = 1.0
    kacc_np = np.zeros((ROW_W, 128), np.float32)
    kacc_np[16:, :] = np.eye(128, dtype=np.float32)
    ksel = jnp.asarray(ksel_np)
    kacc = jnp.asarray(kacc_np)

    cfg_rows = cfg_indices_padded[:, 0]
    cfg_cols = cfg_indices_padded[:, 1]
    gkt_rows = gkt_indices_padded[:, 0]
    gkt_cols = gkt_indices_padded[:, 1]

    att1, v1, sk1 = _dense1(node_fts, hidden, wa_nf, wa_h, ba, wm_nf, wm_h,
                            bm, ws_nf, ws_h, bs)
    sc1 = _edge_layer_sc(att1, v1, cfg_rows, cfg_cols, None)

    ae = _edge_att(gkt_edge_fts, wbig, bae128)
    att2, v2, sk2 = _combine2(sc1, node_fts, sk1, ksel, kacc, wa_nf, wa_h, ba,
                              wm_nf, wm_h, bm, ws_nf, ws_h, bs)
    sc2 = _edge_layer_sc(att2, v2, gkt_rows, gkt_cols, ae)
    return _final(sc2, sk2, ksel, kacc)
